# Initial kernel scaffold; baseline (speedup 1.0000x reference)
import jax
import jax.numpy as jnp
from jax.experimental import pallas as pl


def kernel(x, table):
    raise NotImplementedError("write your pallas kernel here")


# speedup vs baseline: 1.1917x; 1.1917x over previous
import functools

import jax
import jax.numpy as jnp
from jax import lax
from jax.experimental import pallas as pl
from jax.experimental.pallas import tpu as pltpu
from jax.experimental.pallas import tpu_sc as plsc

_D = 64
_NC = 2
_NS = 16
_NW = _NC * _NS
_CHUNK = 128


def _make_lookup(B):
  b_per_w = B // _NW
  n_chunk = b_per_w // _CHUNK
  assert b_per_w % _CHUNK == 0

  mesh = plsc.VectorSubcoreMesh(core_axis_name="c", subcore_axis_name="s")

  @functools.partial(
      pl.kernel,
      mesh=mesh,
      out_type=jax.ShapeDtypeStruct((B, _D), jnp.float32),
      scratch_types=[
          pltpu.VMEM((n_chunk, _CHUNK), jnp.int32),
          pltpu.VMEM((2, _CHUNK, _D), jnp.float32),
          pltpu.SemaphoreType.DMA,
      ],
      compiler_params=pltpu.CompilerParams(use_tc_tiling_on_sc=False),
  )
  def lookup(table_hbm, idx_hbm, out_hbm, idx_v, rows_v, sem):
    wid = lax.axis_index("s") * _NC + lax.axis_index("c")
    base = wid * b_per_w
    pltpu.sync_copy(idx_hbm.at[wid], idx_v)
    for j in range(n_chunk):
      buf = j % 2
      pltpu.async_copy(table_hbm.at[idx_v.at[j]], rows_v.at[buf], sem).wait()
      pltpu.sync_copy(rows_v.at[buf], out_hbm.at[pl.ds(base + j * _CHUNK, _CHUNK)])

  return lookup


def kernel(x, table):
  B = x.shape[0] * x.shape[1]
  idx = x.reshape(_NW, B // _NW // _CHUNK, _CHUNK).astype(jnp.int32)
  out = _make_lookup(B)(table, idx)
  return out.reshape(x.shape + (table.shape[1],))


# speedup vs baseline: 1.3453x; 1.1288x over previous
import functools

import jax
import jax.numpy as jnp
from jax import lax
from jax.experimental import pallas as pl
from jax.experimental.pallas import tpu as pltpu
from jax.experimental.pallas import tpu_sc as plsc

_D = 64
_NC = 2
_NS = 16
_NW = _NC * _NS
_CHUNK = 128


def _make_lookup(B):
  b_per_w = B // _NW
  n_chunk = b_per_w // _CHUNK
  assert b_per_w % _CHUNK == 0

  mesh = plsc.VectorSubcoreMesh(core_axis_name="c", subcore_axis_name="s")

  nbuf = 6

  @functools.partial(
      pl.kernel,
      mesh=mesh,
      out_type=jax.ShapeDtypeStruct((B, _D), jnp.float32),
      scratch_types=[
          pltpu.VMEM((n_chunk, _CHUNK), jnp.int32),
          pltpu.VMEM((nbuf, _CHUNK, _D), jnp.float32),
      ] + [pltpu.SemaphoreType.DMA] * (2 * nbuf),
      compiler_params=pltpu.CompilerParams(use_tc_tiling_on_sc=False),
  )
  def lookup(table_hbm, idx_hbm, out_hbm, idx_v, rows_v, *sems):
    gsems, ssems = sems[:nbuf], sems[nbuf:]
    wid = lax.axis_index("s") * _NC + lax.axis_index("c")
    base = wid * b_per_w
    pltpu.sync_copy(idx_hbm.at[wid], idx_v)

    def gather(j):
      return pltpu.async_copy(
          table_hbm.at[idx_v.at[j]], rows_v.at[j % nbuf], gsems[j % nbuf])

    def scatter(j):
      return pltpu.async_copy(
          rows_v.at[j % nbuf],
          out_hbm.at[pl.ds(base + j * _CHUNK, _CHUNK)], ssems[j % nbuf])

    g = {j: gather(j) for j in range(min(nbuf, n_chunk))}
    s = {}
    for j in range(n_chunk):
      if j >= 1 and j - 1 + nbuf < n_chunk:
        s[j - 1].wait()
        g[j - 1 + nbuf] = gather(j - 1 + nbuf)
      g[j].wait()
      s[j] = scatter(j)
    for j in range(max(0, n_chunk - nbuf), n_chunk):
      s[j].wait()

  return lookup


def kernel(x, table):
  B = x.shape[0] * x.shape[1]
  idx = x.reshape(_NW, B // _NW // _CHUNK, _CHUNK).astype(jnp.int32)
  out = _make_lookup(B)(table, idx)
  return out.reshape(x.shape + (table.shape[1],))


# speedup vs baseline: 1.7507x; 1.3014x over previous
import functools

import jax
import jax.numpy as jnp
from jax import lax
from jax.experimental import pallas as pl
from jax.experimental.pallas import tpu as pltpu
from jax.experimental.pallas import tpu_sc as plsc

_ND = 64
_NJ = 26
_NI = 4096
_V = 65536
_NC = 2
_NS = 16
_NW = _NC * _NS
_DPW = _ND // _NW
_L = 16


def _make_lookup():
  mesh = plsc.VectorSubcoreMesh(core_axis_name="c", subcore_axis_name="s")

  @functools.partial(
      pl.kernel,
      mesh=mesh,
      out_type=jax.ShapeDtypeStruct((_NJ, _ND, _NI), jnp.float32),
      scratch_types=[
          pltpu.VMEM((_V,), jnp.float32),
          pltpu.VMEM((_NI,), jnp.int32),
          pltpu.VMEM((_NI,), jnp.float32),
      ],
      compiler_params=pltpu.CompilerParams(
          use_tc_tiling_on_sc=True, needs_layout_passes=False),
  )
  def lookup(tableT_hbm, xT_hbm, out_hbm, drow_v, idx_v, obuf_v):
    wid = lax.axis_index("s") * _NC + lax.axis_index("c")
    for p in range(_DPW):
      d = wid * _DPW + p
      pltpu.sync_copy(tableT_hbm.at[d], drow_v)
      for j in range(_NJ):
        pltpu.sync_copy(xT_hbm.at[j], idx_v)

        def body(i):
          idx16 = idx_v[pl.ds(i * _L, _L)]
          obuf_v[pl.ds(i * _L, _L)] = plsc.load_gather(drow_v, [idx16])

        plsc.parallel_loop(0, _NI // _L, 1, unroll=4)(body)
        pltpu.sync_copy(obuf_v, out_hbm.at[j].at[d])

  return lookup


def kernel(x, table):
  outp = _make_lookup()(table.T, x.T)
  return jnp.transpose(outp, (2, 0, 1))


# speedup vs baseline: 2.8961x; 1.6543x over previous
import functools

import jax
import jax.numpy as jnp
from jax import lax
from jax.experimental import pallas as pl
from jax.experimental.pallas import tpu as pltpu
from jax.experimental.pallas import tpu_sc as plsc

_ND = 64
_NJ = 26
_NI = 4096
_V = 65536
_NC = 2
_NS = 16
_NW = _NC * _NS
_DPW = _ND // _NW
_L = 16


def _make_lookup():
  mesh = plsc.VectorSubcoreMesh(core_axis_name="c", subcore_axis_name="s")

  @functools.partial(
      pl.kernel,
      mesh=mesh,
      out_type=jax.ShapeDtypeStruct((_NJ, _ND, _NI), jnp.float32),
      scratch_types=[
          pltpu.VMEM((_V,), jnp.float32),
          pltpu.VMEM((_NI,), jnp.int32),
          pltpu.VMEM((_NI,), jnp.int32),
          pltpu.VMEM((_NI,), jnp.float32),
          pltpu.VMEM((_NI,), jnp.float32),
          pltpu.VMEM_SHARED((_NJ * _NI,), jnp.int32),
          pltpu.SemaphoreType.DMA,
          pltpu.SemaphoreType.DMA,
          pltpu.SemaphoreType.DMA,
          pltpu.SemaphoreType.DMA,
          pltpu.SemaphoreType.DMA,
      ],
      compiler_params=pltpu.CompilerParams(
          use_tc_tiling_on_sc=True, needs_layout_passes=False),
  )
  def lookup(tableT_hbm, xT_hbm, out_hbm, drow_v, idx0_v, idx1_v, obuf0_v,
             obuf1_v, xsh, tsem, isems0, isems1, osems0, osems1):
    ibufs = (idx0_v, idx1_v)
    obufs = (obuf0_v, obuf1_v)
    isems = (isems0, isems1)
    osems = (osems0, osems1)
    sid = lax.axis_index("s")
    wid = sid * _NC + lax.axis_index("c")

    tstage = pltpu.async_copy(tableT_hbm.at[wid * _DPW], drow_v, tsem)
    @pl.when(sid == 0)
    def _():
      for j in range(_NJ):
        pltpu.sync_copy(xT_hbm.at[j], xsh.at[pl.ds(j * _NI, _NI)])
    plsc.subcore_barrier()

    def fetch_idx(j):
      return pltpu.async_copy(
          xsh.at[pl.ds(j * _NI, _NI)], ibufs[j % 2], isems[j % 2])

    ifetch = {0: fetch_idx(0)}
    tstage.wait()

    owrites = {}
    for p in range(_DPW):
      d = wid * _DPW + p
      if p > 0:
        pltpu.sync_copy(tableT_hbm.at[d], drow_v)
      for j in range(_NJ):
        buf = j % 2
        nj = j + 1 if j + 1 < _NJ else (0 if p + 1 < _DPW else -1)
        if nj >= 0:
          ifetch[nj] = fetch_idx(nj)
        ifetch[j].wait()
        key = (p, j)
        pkey = (p, j - 2) if j >= 2 else ((p - 1, _NJ - 2 + j) if p > 0 else None)
        if pkey is not None and pkey in owrites:
          owrites[pkey].wait()

        ibuf, obuf = ibufs[buf], obufs[buf]

        def body(i, ibuf=ibuf, obuf=obuf):
          idx16 = ibuf[pl.ds(i * _L, _L)]
          obuf[pl.ds(i * _L, _L)] = plsc.load_gather(drow_v, [idx16])

        plsc.parallel_loop(0, _NI // _L, 1, unroll=8)(body)
        owrites[key] = pltpu.async_copy(
            obuf, out_hbm.at[j].at[d], osems[buf])
    owrites[(_DPW - 1, _NJ - 2)].wait()
    owrites[(_DPW - 1, _NJ - 1)].wait()

  return lookup


def kernel(x, table):
  outp = _make_lookup()(table.T, x.T)
  return jnp.transpose(outp, (2, 0, 1))


# speedup vs baseline: 3.9613x; 1.3678x over previous
import functools

import jax
import jax.numpy as jnp
from jax import lax
from jax.experimental import pallas as pl
from jax.experimental.pallas import tpu as pltpu
from jax.experimental.pallas import tpu_sc as plsc

_ND = 64
_NJ = 26
_NI = 4096
_NP = _NI // 2
_V = 65536
_NC = 2
_NS = 16
_NW = _NC * _NS
_DPW = _ND // _NW
_L = 16


def _make_lookup():
  mesh = plsc.VectorSubcoreMesh(core_axis_name="c", subcore_axis_name="s")

  @functools.partial(
      pl.kernel,
      mesh=mesh,
      out_type=jax.ShapeDtypeStruct((_NJ, _ND, _NI), jnp.float32),
      scratch_types=[
          pltpu.VMEM((_V,), jnp.float32),
          pltpu.VMEM((_NP,), jnp.int32),
          pltpu.VMEM((_NP,), jnp.int32),
          pltpu.VMEM((_NI,), jnp.float32),
          pltpu.VMEM((_NI,), jnp.float32),
          pltpu.VMEM_SHARED((_NJ * _NP,), jnp.int32),
          pltpu.SemaphoreType.DMA,
          pltpu.SemaphoreType.DMA,
          pltpu.SemaphoreType.DMA,
          pltpu.SemaphoreType.DMA,
          pltpu.SemaphoreType.DMA,
          pltpu.SemaphoreType.DMA,
      ],
      compiler_params=pltpu.CompilerParams(
          use_tc_tiling_on_sc=True, needs_layout_passes=False),
  )
  def lookup(tableT_hbm, pT_hbm, out_hbm, drow_v, idx0_v, idx1_v, obuf0_v,
             obuf1_v, xsh, tsem, tsem2, isems0, isems1, osems0, osems1):
    ibufs = (idx0_v, idx1_v)
    obufs = (obuf0_v, obuf1_v)
    isems = (isems0, isems1)
    osems = (osems0, osems1)
    sid = lax.axis_index("s")
    wid = sid * _NC + lax.axis_index("c")

    _H = _V // 2

    def stage_drow(d):
      return (
          pltpu.async_copy(tableT_hbm.at[d].at[pl.ds(0, _H)],
                           drow_v.at[pl.ds(0, _H)], tsem),
          pltpu.async_copy(tableT_hbm.at[d].at[pl.ds(_H, _H)],
                           drow_v.at[pl.ds(_H, _H)], tsem2),
      )

    tstage = stage_drow(wid * _DPW)
    ifetch = {
        0: pltpu.async_copy(pT_hbm.at[0], ibufs[0], isems[0]),
        1: pltpu.async_copy(pT_hbm.at[1], ibufs[1], isems[1]),
    }
    pltpu.sync_copy(pT_hbm.at[sid], xsh.at[pl.ds(sid * _NP, _NP)])
    @pl.when(sid < _NJ - _NS)
    def _():
      pltpu.sync_copy(
          pT_hbm.at[sid + _NS], xsh.at[pl.ds((sid + _NS) * _NP, _NP)])
    plsc.subcore_barrier()

    def fetch_idx(j):
      return pltpu.async_copy(
          xsh.at[pl.ds(j * _NP, _NP)], ibufs[j % 2], isems[j % 2])

    tstage[0].wait()
    tstage[1].wait()

    owrites = {}
    for p in range(_DPW):
      d = wid * _DPW + p
      if p > 0:
        c0, c1 = stage_drow(d)
        c0.wait()
        c1.wait()
      for j in range(_NJ):
        buf = j % 2
        nj = j + 1 if j + 1 < _NJ else (0 if p + 1 < _DPW else -1)
        if nj >= 0 and not (p == 0 and nj == 1):
          ifetch[nj] = fetch_idx(nj)
        ifetch[j].wait()
        key = (p, j)
        pkey = (p, j - 2) if j >= 2 else ((p - 1, _NJ - 2 + j) if p > 0 else None)
        if pkey is not None and pkey in owrites:
          owrites[pkey].wait()

        ibuf, obuf = ibufs[buf], obufs[buf]

        def body(i, ibuf=ibuf, obuf=obuf):
          pv = ibuf[pl.ds(i * _L, _L)]
          lo = pv & jnp.int32(0xFFFF)
          hi = lax.shift_right_logical(pv, jnp.int32(16))
          obuf[pl.ds(i * _L, _L)] = plsc.load_gather(drow_v, [lo])
          obuf[pl.ds(_NP + i * _L, _L)] = plsc.load_gather(drow_v, [hi])

        plsc.parallel_loop(0, _NP // _L, 1, unroll=8)(body)
        owrites[key] = pltpu.async_copy(
            obuf, out_hbm.at[j].at[d], osems[buf])
    owrites[(_DPW - 1, _NJ - 2)].wait()
    owrites[(_DPW - 1, _NJ - 1)].wait()

  return lookup


def kernel(x, table):
  xp = x[:_NP] | (x[_NP:] << 16)
  outp = _make_lookup()(table.T, xp.T)
  return jnp.transpose(outp, (2, 0, 1))
